# baseline (speedup 1.0000x reference)
import jax
import jax.numpy as jnp
from jax import lax
from jax.experimental import pallas as pl
from jax.experimental.pallas import tpu as pltpu
from jax.experimental.pallas import tpu_sc as plsc

_BATCH = 16384
_D = 16
_R = 32
_NC = 2
_NS = 16
_NW = _NC * _NS
_BPW = _BATCH // _NW


def _gather_body(table_hbm, idx_hbm, out_hbm, idx_v, rows_v, sem):
    wid = lax.axis_index("s") * _NC + lax.axis_index("c")
    base = wid * _BPW
    pltpu.sync_copy(idx_hbm.at[pl.ds(base, _BPW)], idx_v)
    pltpu.async_copy(table_hbm.at[idx_v], rows_v, sem).wait()
    pltpu.sync_copy(rows_v, out_hbm.at[pl.ds(base, _BPW)])


def _mm_body(u_ref, r_ref, o_ref):
    o_ref[...] = jnp.dot(u_ref[...], r_ref[...],
                         preferred_element_type=jnp.float32)


def kernel(user, entity, user_emb_table, relation_emb):
    del entity
    mesh = plsc.VectorSubcoreMesh(core_axis_name="c", subcore_axis_name="s")
    gather = pl.kernel(
        _gather_body,
        mesh=mesh,
        compiler_params=pltpu.CompilerParams(use_tc_tiling_on_sc=False),
        out_type=jax.ShapeDtypeStruct((_BATCH, _D), jnp.float32),
        scratch_types=[
            pltpu.VMEM((_BPW,), jnp.int32),
            pltpu.VMEM((_BPW, _D), jnp.float32),
            pltpu.SemaphoreType.DMA,
        ],
    )
    u_emb = gather(user_emb_table, user)

    blk = 2048
    out = pl.pallas_call(
        _mm_body,
        out_shape=jax.ShapeDtypeStruct((_BATCH, _R), jnp.float32),
        grid=(_BATCH // blk,),
        in_specs=[
            pl.BlockSpec((blk, _D), lambda i: (i, 0)),
            pl.BlockSpec((_D, _R), lambda i: (0, 0)),
        ],
        out_specs=pl.BlockSpec((blk, _R), lambda i: (i, 0)),
    )(u_emb, relation_emb)
    return out


# speedup vs baseline: 4.6245x; 4.6245x over previous
import jax
import jax.numpy as jnp
from jax import lax
from jax.experimental import pallas as pl
from jax.experimental.pallas import tpu as pltpu
from jax.experimental.pallas import tpu_sc as plsc

_BATCH = 16384
_D = 16
_R = 32
_NC = 2
_NS = 16
_NW = _NC * _NS
_BPW = _BATCH // _NW
_G = _BPW // 16
_SLAB = 128
_NBUF = 32
_NU = 1000000
_TAIL0 = (_NU // _SLAB) * _SLAB
_TAILN = _NU - _TAIL0


def _gather_body(table_t_hbm, idx_hbm, out_hbm, idx_v, ring_v, cols_v, sem):
    wid = lax.axis_index("s") * _NC + lax.axis_index("c")
    base = wid * _BPW
    pltpu.sync_copy(idx_hbm.at[pl.ds(base, _BPW)], idx_v)

    lanes = lax.iota(jnp.int32, _D)

    def fire(u, slot):
        us = jnp.minimum(u, _TAIL0 - 1)
        col0 = pl.multiple_of((us >> 7) * _SLAB, _SLAB)
        return pltpu.async_copy(table_t_hbm.at[:, pl.ds(col0, _SLAB)],
                                ring_v.at[slot], sem)

    def extract(u, i, slot):
        col = jnp.minimum(u, _TAIL0 - 1) & (_SLAB - 1)
        v = plsc.load_gather(
            ring_v,
            [jnp.full((_D,), slot, jnp.int32), lanes,
             jnp.full((_D,), col, jnp.int32)],
        )
        plsc.store_scatter(cols_v, [lanes, jnp.full((_D,), i, jnp.int32)], v)

    def group_body(g, _):
        vec = idx_v[pl.ds(pl.multiple_of(g * 16, 16), 16)]
        copies = [fire(vec[j], j) for j in range(16)]
        for c in copies:
            c.wait()
        for j in range(16):
            extract(vec[j], g * 16 + j, j)
        return 0

    lax.fori_loop(0, _G, group_body, 0)

    pltpu.sync_copy(cols_v, out_hbm.at[:, pl.ds(base, _BPW)])


def _mm_body(u_ref, user_ref, tail_ref, r_ref, o_ref):
    user = user_ref[...]
    t = user - _TAIL0
    onehot = (t[:, None] == lax.iota(jnp.int32, _TAILN)[None, :])
    fixed = lax.dot_general(tail_ref[...], onehot.astype(jnp.float32),
                            (((1,), (1,)), ((), ())),
                            preferred_element_type=jnp.float32)
    u = jnp.where((t >= 0)[None, :], fixed, u_ref[...])
    o_ref[...] = lax.dot_general(r_ref[...], u,
                                 (((0,), (0,)), ((), ())),
                                 preferred_element_type=jnp.float32)


def kernel(user, entity, user_emb_table, relation_emb):
    del entity
    table_t = user_emb_table.T

    mesh = plsc.VectorSubcoreMesh(core_axis_name="c", subcore_axis_name="s")
    gather = pl.kernel(
        _gather_body,
        mesh=mesh,
        compiler_params=pltpu.CompilerParams(use_tc_tiling_on_sc=True,
                                             needs_layout_passes=False),
        out_type=jax.ShapeDtypeStruct((_D, _BATCH), jnp.float32),
        scratch_types=[
            pltpu.VMEM((_BPW,), jnp.int32),
            pltpu.VMEM((16, _D, _SLAB), jnp.float32),
            pltpu.VMEM((_D, _BPW), jnp.float32),
            pltpu.SemaphoreType.DMA,
        ],
    )
    u_embT = gather(table_t, user)

    tail = lax.slice(table_t, (0, _TAIL0), (_D, _NU))

    blk = 2048
    out_t = pl.pallas_call(
        _mm_body,
        out_shape=jax.ShapeDtypeStruct((_R, _BATCH), jnp.float32),
        grid=(_BATCH // blk,),
        in_specs=[
            pl.BlockSpec((_D, blk), lambda i: (0, i)),
            pl.BlockSpec((blk,), lambda i: (i,)),
            pl.BlockSpec((_D, _TAILN), lambda i: (0, 0)),
            pl.BlockSpec((_D, _R), lambda i: (0, 0)),
        ],
        out_specs=pl.BlockSpec((_R, blk), lambda i: (0, i)),
    )(u_embT, user, tail, relation_emb)
    return out_t.T


# speedup vs baseline: 5.4163x; 1.1712x over previous
import jax
import jax.numpy as jnp
from jax import lax
from jax.experimental import pallas as pl
from jax.experimental.pallas import tpu as pltpu
from jax.experimental.pallas import tpu_sc as plsc

_BATCH = 16384
_D = 16
_R = 32
_NC = 2
_NS = 16
_NW = _NC * _NS
_BPW = _BATCH // _NW
_G = _BPW // 16
_SLAB = 128
_NBUF = 32
_NU = 1000000
_TAIL0 = (_NU // _SLAB) * _SLAB
_TAILN = _NU - _TAIL0


def _gather_body(table_t_hbm, idx_hbm, out_hbm, idx_v, ring_v, cols_v,
                 sem_a, sem_b):
    wid = lax.axis_index("s") * _NC + lax.axis_index("c")
    base = wid * _BPW
    pltpu.sync_copy(idx_hbm.at[pl.ds(base, _BPW)], idx_v)

    lanes = lax.iota(jnp.int32, _D)

    def fire(u, slot, sem):
        us = jnp.minimum(u, _TAIL0 - 1)
        col0 = pl.multiple_of((us >> 7) * _SLAB, _SLAB)
        return pltpu.async_copy(table_t_hbm.at[:, pl.ds(col0, _SLAB)],
                                ring_v.at[slot], sem)

    def extract(u, i, slot):
        col = jnp.minimum(u, _TAIL0 - 1) & (_SLAB - 1)
        v = plsc.load_gather(
            ring_v,
            [jnp.full((_D,), slot, jnp.int32), lanes,
             jnp.full((_D,), col, jnp.int32)],
        )
        plsc.store_scatter(cols_v, [lanes, jnp.full((_D,), i, jnp.int32)], v)

    def load_group(g):
        return idx_v[pl.ds(pl.multiple_of(g * 16, 16), 16)]

    def fire_group(g, half, sem):
        vec = load_group(g)
        for j in range(16):
            fire(vec[j], half * 16 + j, sem)

    def drain(half, sem):
        for j in range(16):
            pltpu.make_async_copy(table_t_hbm.at[:, pl.ds(0, _SLAB)],
                                  ring_v.at[half * 16 + j], sem).wait()

    def extract_group(g, half):
        vec = load_group(g)
        for j in range(16):
            extract(vec[j], g * 16 + j, half * 16 + j)

    fire_group(0, 0, sem_a)
    fire_group(1, 1, sem_b)

    def pair_body(p, _):
        g = p * 2
        drain(0, sem_a)
        extract_group(g, 0)
        fire_group(g + 2, 0, sem_a)
        drain(1, sem_b)
        extract_group(g + 1, 1)
        fire_group(g + 3, 1, sem_b)
        return 0

    lax.fori_loop(0, _G // 2 - 1, pair_body, 0)

    drain(0, sem_a)
    extract_group(_G - 2, 0)
    drain(1, sem_b)
    extract_group(_G - 1, 1)

    pltpu.sync_copy(cols_v, out_hbm.at[:, pl.ds(base, _BPW)])


def _mm_body(u_ref, user_ref, tail_ref, r_ref, o_ref):
    user = user_ref[...]
    t = user - _TAIL0
    onehot = (t[:, None] == lax.iota(jnp.int32, _TAILN)[None, :])
    fixed = lax.dot_general(tail_ref[...], onehot.astype(jnp.float32),
                            (((1,), (1,)), ((), ())),
                            preferred_element_type=jnp.float32)
    u = jnp.where((t >= 0)[None, :], fixed, u_ref[...])
    o_ref[...] = lax.dot_general(r_ref[...], u,
                                 (((0,), (0,)), ((), ())),
                                 preferred_element_type=jnp.float32)


def kernel(user, entity, user_emb_table, relation_emb):
    del entity
    table_t = user_emb_table.T

    mesh = plsc.VectorSubcoreMesh(core_axis_name="c", subcore_axis_name="s")
    gather = pl.kernel(
        _gather_body,
        mesh=mesh,
        compiler_params=pltpu.CompilerParams(use_tc_tiling_on_sc=True,
                                             needs_layout_passes=False),
        out_type=jax.ShapeDtypeStruct((_D, _BATCH), jnp.float32),
        scratch_types=[
            pltpu.VMEM((_BPW,), jnp.int32),
            pltpu.VMEM((32, _D, _SLAB), jnp.float32),
            pltpu.VMEM((_D, _BPW), jnp.float32),
            pltpu.SemaphoreType.DMA,
            pltpu.SemaphoreType.DMA,
        ],
    )
    u_embT = gather(table_t, user)

    tail = lax.slice(table_t, (0, _TAIL0), (_D, _NU))

    blk = 2048
    out_t = pl.pallas_call(
        _mm_body,
        out_shape=jax.ShapeDtypeStruct((_R, _BATCH), jnp.float32),
        grid=(_BATCH // blk,),
        in_specs=[
            pl.BlockSpec((_D, blk), lambda i: (0, i)),
            pl.BlockSpec((blk,), lambda i: (i,)),
            pl.BlockSpec((_D, _TAILN), lambda i: (0, 0)),
            pl.BlockSpec((_D, _R), lambda i: (0, 0)),
        ],
        out_specs=pl.BlockSpec((_R, blk), lambda i: (0, i)),
    )(u_embT, user, tail, relation_emb)
    return out_t.T


# speedup vs baseline: 5.6407x; 1.0414x over previous
import jax
import jax.numpy as jnp
from jax import lax
from jax.experimental import pallas as pl
from jax.experimental.pallas import tpu as pltpu
from jax.experimental.pallas import tpu_sc as plsc

_BATCH = 16384
_D = 16
_R = 32
_NC = 2
_NS = 16
_NW = _NC * _NS
_BPW = _BATCH // _NW
_G = _BPW // 16
_SLAB = 128
_NBUF = 32
_NU = 1000000
_TAIL0 = (_NU // _SLAB) * _SLAB
_TAILN = _NU - _TAIL0


def _gather_body(table_t_hbm, idx_hbm, out_hbm, idx_v, ring_v, cols_v,
                 sem_a, sem_b, sem_c):
    wid = lax.axis_index("s") * _NC + lax.axis_index("c")
    base = wid * _BPW
    pltpu.sync_copy(idx_hbm.at[pl.ds(base, _BPW)], idx_v)

    lanes = lax.iota(jnp.int32, _D)

    def fire(u, slot, sem):
        us = jnp.minimum(u, _TAIL0 - 1)
        col0 = pl.multiple_of((us >> 7) * _SLAB, _SLAB)
        return pltpu.async_copy(table_t_hbm.at[:, pl.ds(col0, _SLAB)],
                                ring_v.at[slot], sem)

    def extract(u, i, slot):
        col = jnp.minimum(u, _TAIL0 - 1) & (_SLAB - 1)
        v = plsc.load_gather(
            ring_v,
            [jnp.full((_D,), slot, jnp.int32), lanes,
             jnp.full((_D,), col, jnp.int32)],
        )
        plsc.store_scatter(cols_v, [lanes, jnp.full((_D,), i, jnp.int32)], v)

    def load_group(g):
        return idx_v[pl.ds(pl.multiple_of(g * 16, 16), 16)]

    def fire_group(g, half, sem):
        vec = load_group(g)
        for j in range(16):
            fire(vec[j], half * 16 + j, sem)

    def drain(half, sem):
        for j in range(16):
            pltpu.make_async_copy(table_t_hbm.at[:, pl.ds(0, _SLAB)],
                                  ring_v.at[half * 16 + j], sem).wait()

    def extract_group(g, half):
        vec = load_group(g)
        for j in range(16):
            extract(vec[j], g * 16 + j, half * 16 + j)

    fire_group(0, 0, sem_a)
    fire_group(1, 1, sem_b)
    fire_group(2, 2, sem_c)

    def triple_body(p, _):
        g = p * 3
        drain(0, sem_a)
        extract_group(g, 0)
        fire_group(g + 3, 0, sem_a)
        drain(1, sem_b)
        extract_group(g + 1, 1)
        fire_group(g + 4, 1, sem_b)
        drain(2, sem_c)
        extract_group(g + 2, 2)
        fire_group(g + 5, 2, sem_c)
        return 0

    lax.fori_loop(0, 9, triple_body, 0)

    drain(0, sem_a)
    extract_group(27, 0)
    fire_group(30, 0, sem_a)
    drain(1, sem_b)
    extract_group(28, 1)
    fire_group(31, 1, sem_b)
    drain(2, sem_c)
    extract_group(29, 2)
    drain(0, sem_a)
    extract_group(30, 0)
    drain(1, sem_b)
    extract_group(31, 1)

    pltpu.sync_copy(cols_v, out_hbm.at[:, pl.ds(base, _BPW)])


def _mm_body(u_ref, user_ref, tail_ref, r_ref, o_ref):
    user = user_ref[...]
    t = user - _TAIL0
    onehot = (t[:, None] == lax.iota(jnp.int32, _TAILN)[None, :])
    fixed = lax.dot_general(tail_ref[...], onehot.astype(jnp.float32),
                            (((1,), (1,)), ((), ())),
                            preferred_element_type=jnp.float32)
    u = jnp.where((t >= 0)[None, :], fixed, u_ref[...])
    o_ref[...] = lax.dot_general(r_ref[...], u,
                                 (((0,), (0,)), ((), ())),
                                 preferred_element_type=jnp.float32)


def kernel(user, entity, user_emb_table, relation_emb):
    del entity
    table_t = user_emb_table.T

    mesh = plsc.VectorSubcoreMesh(core_axis_name="c", subcore_axis_name="s")
    gather = pl.kernel(
        _gather_body,
        mesh=mesh,
        compiler_params=pltpu.CompilerParams(use_tc_tiling_on_sc=True,
                                             needs_layout_passes=False),
        out_type=jax.ShapeDtypeStruct((_D, _BATCH), jnp.float32),
        scratch_types=[
            pltpu.VMEM((_BPW,), jnp.int32),
            pltpu.VMEM((48, _D, _SLAB), jnp.float32),
            pltpu.VMEM((_D, _BPW), jnp.float32),
            pltpu.SemaphoreType.DMA,
            pltpu.SemaphoreType.DMA,
            pltpu.SemaphoreType.DMA,
        ],
    )
    u_embT = gather(table_t, user)

    tail = lax.slice(table_t, (0, _TAIL0), (_D, _NU))

    blk = 2048
    out_t = pl.pallas_call(
        _mm_body,
        out_shape=jax.ShapeDtypeStruct((_R, _BATCH), jnp.float32),
        grid=(_BATCH // blk,),
        in_specs=[
            pl.BlockSpec((_D, blk), lambda i: (0, i)),
            pl.BlockSpec((blk,), lambda i: (i,)),
            pl.BlockSpec((_D, _TAILN), lambda i: (0, 0)),
            pl.BlockSpec((_D, _R), lambda i: (0, 0)),
        ],
        out_specs=pl.BlockSpec((_R, blk), lambda i: (0, i)),
    )(u_embT, user, tail, relation_emb)
    return out_t.T
